# baseline (speedup 1.0000x reference)
import functools

import jax
import jax.numpy as jnp
from jax.experimental import pallas as pl
from jax.experimental.pallas import tpu as pltpu

VOCAB = 8192
D = 8192
B, T = 16, 512
N_IDX = B * T
RPB = 8


def _tc_body(idx_ref, *refs):
    outs = refs[RPB]
    for j in range(RPB):
        outs[j * 1:(j + 1) * 1, :] = refs[j][...]


def _tc_body2(idx_ref, *refs):
    out = refs[RPB]
    for j in range(RPB):
        out[j, 0, :] = refs[j][0, 0, :]


@jax.jit
def _gather_tc(idx_flat, table):
    grid = (N_IDX // RPB,)

    def in_map(j):
        return lambda i, idx_ref: (idx_ref[i * RPB + j], 0, 0)

    grid_spec = pltpu.PrefetchScalarGridSpec(
        num_scalar_prefetch=1,
        grid=grid,
        in_specs=[
            pl.BlockSpec((1, 1, D), in_map(j)) for j in range(RPB)
        ],
        out_specs=pl.BlockSpec((RPB, 1, D), lambda i, idx_ref: (i, 0, 0)),
    )
    return pl.pallas_call(
        _tc_body2,
        grid_spec=grid_spec,
        out_shape=jax.ShapeDtypeStruct((N_IDX, 1, D), jnp.float32),
    )(idx_flat, *([table.reshape(VOCAB, 1, D)] * RPB))


def kernel(idx, table):
    idx_flat = idx.reshape(-1).astype(jnp.int32)
    out = _gather_tc(idx_flat, table)
    return out.reshape(B, T, D)
